# baseline (speedup 1.0000x reference)
import jax
import jax.numpy as jnp
from jax import lax
from jax.experimental import pallas as pl
from jax.experimental.pallas import tpu as pltpu
from jax.experimental.pallas import tpu_sc as plsc

N = 10000
E = 320000
H = 8
F = 16
D = 128

NP = 10240
EP = 327680
NTILES = 32
CHUNK = 32
GROUP = 1
CHUNKS_PER_TILE = EP // (NTILES * CHUNK)
ROWS_PER_TILE = NP // 16
DROWS_PER_TILE = NP // 128

_SELU_ALPHA = 1.6732632423543772
_SELU_SCALE = 1.0507009873554805


_BR = 256
_NBLK = NP // _BR


def _dotT(a, b):
  return lax.dot_general(
      a, b, (((1,), (0,)), ((), ())),
      preferred_element_type=jnp.float32,
      precision=lax.Precision.HIGHEST,
  )


def _pack_outputs(feat, elr, erl, feat_ref, elx_ref, erx_ref):
  z = jnp.zeros((feat.shape[0], 112), jnp.float32)
  feat_ref[...] = feat
  elx_ref[...] = jnp.concatenate([elr, z], axis=1)
  erx_ref[...] = jnp.concatenate([erl, z], axis=1)


def _tc1_body(x_ref, w_ref, c1_ref, c2_ref, feat_ref, elx_ref, erx_ref):
  feat = _dotT(x_ref[...], w_ref[...])
  elr = _dotT(feat, c1_ref[...])
  erl = _dotT(feat, c2_ref[...])
  _pack_outputs(feat, elr, erl, feat_ref, elx_ref, erx_ref)


def _selu(x):
  return _SELU_SCALE * jnp.where(
      x > 0, x, _SELU_ALPHA * (jnp.exp(jnp.minimum(x, 0.0)) - 1.0))


def _mid_body(u0_ref, u1_ref, d0_ref, d1_ref, b16_ref, bias_ref,
              w_ref, c1_ref, c2_ref, feat_ref, elx_ref, erx_ref):
  u = u0_ref[...] + u1_ref[...]
  d = d0_ref[...] + d1_ref[...]
  r = jnp.where(d > 0, 1.0 / d, 0.0)
  d128 = _dotT(r, b16_ref[...])
  h = _selu(u * d128 + bias_ref[...])
  feat = _dotT(h, w_ref[...])
  elr = _dotT(feat, c1_ref[...])
  erl = _dotT(feat, c2_ref[...])
  _pack_outputs(feat, elr, erl, feat_ref, elx_ref, erx_ref)


def _final_body(u0_ref, u1_ref, d0_ref, d1_ref, b16_ref, bias_ref, mask_ref,
                out_ref):
  u = u0_ref[...] + u1_ref[...]
  d = d0_ref[...] + d1_ref[...]
  r = jnp.where(d > 0, 1.0 / d, 0.0)
  d128 = _dotT(r, b16_ref[...])
  h = _selu(u * d128 + bias_ref[...])
  p = jnp.sum(h * mask_ref[...]) * (1.0 / (128.0 * N))
  i = pl.program_id(0)
  prev = jnp.where(i == 0, 0.0, out_ref[0, 0])
  out_ref[0, 0] = prev + p


def _row_spec(width):
  return pl.BlockSpec((_BR, width), lambda i: (i, 0))


def _rep_spec(shape):
  nd = len(shape)
  return pl.BlockSpec(shape, lambda i: (0,) * nd)


def _tc_layer1(x, w, c1, c2):
  return pl.pallas_call(
      _tc1_body,
      grid=(_NBLK,),
      in_specs=[_row_spec(128), _rep_spec((128, 128)), _rep_spec((128, 16)),
                _rep_spec((128, 16))],
      out_specs=[_row_spec(128), _row_spec(128), _row_spec(128)],
      out_shape=[
          jax.ShapeDtypeStruct((NP, 128), jnp.float32),
          jax.ShapeDtypeStruct((NP, 128), jnp.float32),
          jax.ShapeDtypeStruct((NP, 128), jnp.float32),
      ],
  )(x, w, c1, c2)


def _tc_mid(u0, u1, d0, d1, b16, bias, w, c1, c2):
  return pl.pallas_call(
      _mid_body,
      grid=(_NBLK,),
      in_specs=[_row_spec(128), _row_spec(128), _row_spec(16), _row_spec(16),
                _rep_spec((16, 128)), _rep_spec((1, 128)),
                _rep_spec((128, 128)), _rep_spec((128, 16)),
                _rep_spec((128, 16))],
      out_specs=[_row_spec(128), _row_spec(128), _row_spec(128)],
      out_shape=[
          jax.ShapeDtypeStruct((NP, 128), jnp.float32),
          jax.ShapeDtypeStruct((NP, 128), jnp.float32),
          jax.ShapeDtypeStruct((NP, 128), jnp.float32),
      ],
  )(u0, u1, d0, d1, b16, bias, w, c1, c2)


def _tc_final(u0, u1, d0, d1, b16, bias, mask2d):
  return pl.pallas_call(
      _final_body,
      grid=(_NBLK,),
      in_specs=[_row_spec(128), _row_spec(128), _row_spec(16), _row_spec(16),
                _rep_spec((16, 128)), _rep_spec((1, 128)),
                _row_spec(1)],
      out_specs=pl.BlockSpec(memory_space=pltpu.SMEM),
      out_shape=jax.ShapeDtypeStruct((1, 1), jnp.float32),
  )(u0, u1, d0, d1, b16, bias, mask2d)


NTOT = NP + NP // 8
TROWS_PER_TILE = NTOT // 16


def _edge_body(src_hbm, dst_hbm, feat_hbm, elx_hbm, erx_hbm,
               out_all, acc, sidx0, sidx1, dall0, dall1,
               me0, me1, elg, erg, eeb, gsem, ssem):
  c = lax.axis_index("c")
  s = lax.axis_index("s")
  wid = c * 16 + s
  sidx = [sidx0, sidx1]
  dall = [dall0, dall1]
  me = [me0, me1]
  z16 = jnp.zeros((16,), jnp.float32)
  zi16 = jnp.zeros((16,), jnp.int32)
  iota16 = lax.iota(jnp.int32, 16)

  def _zero_rows(i, carry):
    for k in range(8):
      me0[i, pl.ds(k * 16, 16)] = z16
      me1[i, pl.ds(k * 16, 16)] = z16
    return carry

  lax.fori_loop(0, 2 * CHUNK, _zero_rows, 0)
  for k in range(2 * CHUNK // 16):
    dall0[pl.ds(k * 16, 16)] = zi16
    dall1[pl.ds(k * 16, 16)] = zi16

  row0 = s * TROWS_PER_TILE
  for off in range(0, TROWS_PER_TILE, 2 * CHUNK):
    sz = min(2 * CHUNK, TROWS_PER_TILE - off)
    pltpu.sync_copy(me0.at[pl.ds(0, sz)], acc.at[pl.ds(row0 + off, sz)])
  plsc.subcore_barrier()

  edge0 = wid * CHUNKS_PER_TILE * CHUNK

  def _fetch(ch, p):
    off = edge0 + ch * CHUNK
    pltpu.sync_copy(src_hbm.at[pl.ds(off, CHUNK)], sidx[p])
    pltpu.sync_copy(dst_hbm.at[pl.ds(off, CHUNK)],
                    dall[p].at[pl.ds(0, CHUNK)])
    pltpu.async_copy(feat_hbm.at[sidx[p]], me[p].at[pl.ds(0, CHUNK)], gsem)
    pltpu.async_copy(elx_hbm.at[sidx[p]], elg, gsem)
    pltpu.async_copy(erx_hbm.at[dall[p].at[pl.ds(0, CHUNK)]], erg, gsem)

  def _wait_fetch(p):
    pltpu.make_async_copy(feat_hbm.at[sidx[p]],
                          me[p].at[pl.ds(0, CHUNK)], gsem).wait()
    pltpu.make_async_copy(elx_hbm.at[sidx[p]], elg, gsem).wait()
    pltpu.make_async_copy(erx_hbm.at[dall[p].at[pl.ds(0, CHUNK)]],
                          erg, gsem).wait()

  def _scatter(p):
    pltpu.async_copy(me[p], acc.at[dall[p]], ssem, add=True)

  def _wait_scatter_rezero(p):
    pltpu.make_async_copy(me[p], acc.at[dall[p]], ssem).wait()
    dall_p, me_p = dall[p], me[p]
    for k in range(CHUNK // 16):
      rows = iota16 + (CHUNK + k * 16)
      dv = dall_p[pl.ds(k * 16, 16)]
      colb = (dv & 7) * 16
      for h in range(H):
        plsc.store_scatter(me_p, [rows, colb + h], z16)

  def _ee_phase(p):
    def _edge(i, inner):
      sv = elg[i, pl.ds(0, 16)] + erg[i, pl.ds(0, 16)]
      e = jnp.maximum(sv, 0.2 * sv)
      eeb[i] = jnp.exp(e)
      return inner

    lax.fori_loop(0, CHUNK, _edge, 0)

  def _scale_pack_phase(p):
    me_p, dall_p = me[p], dall[p]

    def _edge(i, inner):
      ee = eeb[i]
      for h in range(H):
        idx = jnp.full((16,), h, dtype=jnp.int32)
        bc = jnp.take_along_axis(
            ee, idx, axis=0,
            mode=lax.GatherScatterMode.PROMISE_IN_BOUNDS)
        fsl = pl.ds(h * 16, 16)
        me_p[i, fsl] = bc * me_p[i, fsl]
      return inner

    lax.fori_loop(0, CHUNK, _edge, 0)

    for k in range(CHUNK // 16):
      rows = iota16 + (k * 16)
      dv = dall_p[pl.ds(k * 16, 16)]
      dall_p[pl.ds(CHUNK + k * 16, 16)] = (
          lax.shift_right_logical(dv, 3) + NP)
      colb = (dv & 7) * 16
      for h in range(H):
        vals = plsc.load_gather(
            eeb, [rows, jnp.full((16,), h, dtype=jnp.int32)])
        plsc.store_scatter(me_p, [rows + CHUNK, colb + h], vals)

  _fetch(jnp.int32(0), 0)
  _scatter(1)

  def _body(g2, carry):
    ch0 = g2 * 2
    for p in range(2):
      ch = ch0 + p
      _wait_fetch(p)
      _ee_phase(p)
      _wait_scatter_rezero(1 - p)
      _fetch(jnp.minimum(ch + 1, CHUNKS_PER_TILE - 1), 1 - p)
      _scale_pack_phase(p)
      _scatter(p)
    return carry

  lax.fori_loop(0, CHUNKS_PER_TILE // 2, _body, 0)
  _wait_fetch(0)
  pltpu.make_async_copy(me1, acc.at[dall1], ssem).wait()

  plsc.subcore_barrier()
  rows = pl.ds(row0, TROWS_PER_TILE)
  pltpu.sync_copy(acc.at[rows], out_all.at[c, rows])


def _edge_pass(srcf, dstf, feat, elx, erx):
  mesh = plsc.VectorSubcoreMesh(core_axis_name="c", subcore_axis_name="s")
  idx_t = pltpu.VMEM((CHUNK,), jnp.int32)
  idx2_t = pltpu.VMEM((2 * CHUNK,), jnp.int32)
  row_t = pltpu.VMEM((CHUNK, 128), jnp.float32)
  row2_t = pltpu.VMEM((2 * CHUNK, 128), jnp.float32)
  ee_t = pltpu.VMEM((CHUNK, 16), jnp.float32)
  out = pl.kernel(
      _edge_body,
      out_type=jax.ShapeDtypeStruct((2, NTOT, 128), jnp.float32),
      mesh=mesh,
      compiler_params=pltpu.CompilerParams(needs_layout_passes=False),
      scratch_types=[
          pltpu.VMEM_SHARED((NTOT, 128), jnp.float32),
          idx_t, idx_t,
          idx2_t, idx2_t,
          row2_t, row2_t,
          row_t,
          row_t,
          ee_t,
          pltpu.SemaphoreType.DMA,
          pltpu.SemaphoreType.DMA,
      ],
  )(srcf, dstf, feat, elx, erx)
  u = out[:, :NP, :]
  d = out[:, NP:, :].reshape(2, NP, 16)
  return u, d


def _attn_mats(al, ar):
  a = al.reshape(H, F)
  r = ar.reshape(H, F)
  eye = jnp.eye(H, dtype=jnp.float32)
  cl = (a[:, :, None] * eye[:, None, :]).reshape(H * F, H)
  cr = (r[:, :, None] * eye[:, None, :]).reshape(H * F, H)
  c1 = jnp.concatenate([cl, cr], axis=1)
  c2 = jnp.concatenate([cr, cl], axis=1)
  return c1, c2


@jax.jit
def kernel(x, edge_index, mask, W1, al1, ar1, b1, W2, al2, ar2, b2):
  xp = jnp.pad(x, ((0, NP - N), (0, 0)))
  src = edge_index[0]
  dst = edge_index[1]
  npad = EP - E
  pad_idx = N + (jnp.arange(npad, dtype=jnp.int32) % (NP - N))
  srcf = jnp.concatenate([src, pad_idx])
  dstf = jnp.concatenate([dst, pad_idx])

  w1t = W1.T
  w2t = W2.T
  c11, c12 = _attn_mats(al1, ar1)
  c21, c22 = _attn_mats(al2, ar2)
  b16 = jnp.concatenate(
      [jnp.kron(jnp.eye(8, dtype=jnp.float32), jnp.ones((1, 16), jnp.float32)),
       jnp.zeros((8, 128), jnp.float32)], axis=0)
  bias1 = b1.reshape(1, 128)
  bias2 = b2.reshape(1, 128)
  mask2d = jnp.pad(mask, (0, NP - N)).reshape(NP, 1)

  feat1, elx1, erx1 = _tc_layer1(xp, w1t, c11, c12)
  u1, d1 = _edge_pass(srcf, dstf, feat1, elx1, erx1)

  feat2, elx2, erx2 = _tc_mid(u1[0], u1[1], d1[0], d1[1], b16, bias1,
                              w2t, c21, c22)
  u2, d2 = _edge_pass(srcf, dstf, feat2, elx2, erx2)

  res = _tc_final(u2[0], u2[1], d2[0], d2[1], b16, bias2, mask2d)
  return res[0, 0]


# speedup vs baseline: 1.2447x; 1.2447x over previous
import jax
import jax.numpy as jnp
from jax import lax
from jax.experimental import pallas as pl
from jax.experimental.pallas import tpu as pltpu
from jax.experimental.pallas import tpu_sc as plsc

N = 10000
E = 320000
H = 8
F = 16
D = 128

NP = 10240
EP = 327680
NTILES = 32
CHUNK = 32
GROUP = 1
CHUNKS_PER_TILE = EP // (NTILES * CHUNK)
ROWS_PER_TILE = NP // 16
DROWS_PER_TILE = NP // 128

_SELU_ALPHA = 1.6732632423543772
_SELU_SCALE = 1.0507009873554805


_BR = 256
_NBLK = NP // _BR


def _dotT(a, b):
  return lax.dot_general(
      a, b, (((1,), (0,)), ((), ())),
      preferred_element_type=jnp.float32,
      precision=lax.Precision.HIGHEST,
  )


def _pack_outputs(feat, elr, erl, feat_ref, elx_ref, erx_ref):
  z = jnp.zeros((feat.shape[0], 112), jnp.float32)
  feat_ref[...] = feat
  elx_ref[...] = jnp.concatenate([elr, z], axis=1)
  erx_ref[...] = jnp.concatenate([erl, z], axis=1)


def _tc1_body(x_ref, w_ref, c1_ref, c2_ref, feat_ref, elx_ref, erx_ref):
  feat = _dotT(x_ref[...], w_ref[...])
  elr = _dotT(feat, c1_ref[...])
  erl = _dotT(feat, c2_ref[...])
  _pack_outputs(feat, elr, erl, feat_ref, elx_ref, erx_ref)


def _selu(x):
  return _SELU_SCALE * jnp.where(
      x > 0, x, _SELU_ALPHA * (jnp.exp(jnp.minimum(x, 0.0)) - 1.0))


def _mid_body(u0_ref, u1_ref, d0_ref, d1_ref, b16_ref, bias_ref,
              w_ref, c1_ref, c2_ref, feat_ref, elx_ref, erx_ref):
  u = u0_ref[...] + u1_ref[...]
  d = d0_ref[...] + d1_ref[...]
  r = jnp.where(d > 0, 1.0 / d, 0.0)
  d128 = _dotT(r, b16_ref[...])
  h = _selu(u * d128 + bias_ref[...])
  feat = _dotT(h, w_ref[...])
  elr = _dotT(feat, c1_ref[...])
  erl = _dotT(feat, c2_ref[...])
  _pack_outputs(feat, elr, erl, feat_ref, elx_ref, erx_ref)


def _final_body(u0_ref, u1_ref, d0_ref, d1_ref, b16_ref, bias_ref, mask_ref,
                out_ref):
  u = u0_ref[...] + u1_ref[...]
  d = d0_ref[...] + d1_ref[...]
  r = jnp.where(d > 0, 1.0 / d, 0.0)
  d128 = _dotT(r, b16_ref[...])
  h = _selu(u * d128 + bias_ref[...])
  p = jnp.sum(h * mask_ref[...]) * (1.0 / (128.0 * N))
  i = pl.program_id(0)
  prev = jnp.where(i == 0, 0.0, out_ref[0, 0])
  out_ref[0, 0] = prev + p


def _row_spec(width):
  return pl.BlockSpec((_BR, width), lambda i: (i, 0))


def _rep_spec(shape):
  nd = len(shape)
  return pl.BlockSpec(shape, lambda i: (0,) * nd)


def _tc_layer1(x, w, c1, c2):
  return pl.pallas_call(
      _tc1_body,
      grid=(_NBLK,),
      in_specs=[_row_spec(128), _rep_spec((128, 128)), _rep_spec((128, 16)),
                _rep_spec((128, 16))],
      out_specs=[_row_spec(128), _row_spec(128), _row_spec(128)],
      out_shape=[
          jax.ShapeDtypeStruct((NP, 128), jnp.float32),
          jax.ShapeDtypeStruct((NP, 128), jnp.float32),
          jax.ShapeDtypeStruct((NP, 128), jnp.float32),
      ],
  )(x, w, c1, c2)


def _tc_mid(u0, u1, d0, d1, b16, bias, w, c1, c2):
  return pl.pallas_call(
      _mid_body,
      grid=(_NBLK,),
      in_specs=[_row_spec(128), _row_spec(128), _row_spec(16), _row_spec(16),
                _rep_spec((16, 128)), _rep_spec((1, 128)),
                _rep_spec((128, 128)), _rep_spec((128, 16)),
                _rep_spec((128, 16))],
      out_specs=[_row_spec(128), _row_spec(128), _row_spec(128)],
      out_shape=[
          jax.ShapeDtypeStruct((NP, 128), jnp.float32),
          jax.ShapeDtypeStruct((NP, 128), jnp.float32),
          jax.ShapeDtypeStruct((NP, 128), jnp.float32),
      ],
  )(u0, u1, d0, d1, b16, bias, w, c1, c2)


def _tc_final(u0, u1, d0, d1, b16, bias, mask2d):
  return pl.pallas_call(
      _final_body,
      grid=(_NBLK,),
      in_specs=[_row_spec(128), _row_spec(128), _row_spec(16), _row_spec(16),
                _rep_spec((16, 128)), _rep_spec((1, 128)),
                _row_spec(1)],
      out_specs=pl.BlockSpec(memory_space=pltpu.SMEM),
      out_shape=jax.ShapeDtypeStruct((1, 1), jnp.float32),
  )(u0, u1, d0, d1, b16, bias, mask2d)


def _edge_body(src_hbm, dst_hbm, feat_hbm, elx_hbm, erx_hbm,
               out_u, out_d, acc_u, acc_d,
               sidx0, sidx1, didx0, didx1, didx8,
               msg0, msg1, elg0, elg1, erg0, erg1, eeb, eep,
               gsem0, gsem1, usem, dsem):
  c = lax.axis_index("c")
  s = lax.axis_index("s")
  wid = c * 16 + s
  sidx = [sidx0, sidx1]
  didx = [didx0, didx1]
  msg = [msg0, msg1]
  elg = [elg0, elg1]
  erg = [erg0, erg1]
  gsem = [gsem0, gsem1]
  z16 = jnp.zeros((16,), jnp.float32)

  def _zero_rows(i, carry):
    for k in range(8):
      msg0[i, pl.ds(k * 16, 16)] = z16
      eep[i, pl.ds(k * 16, 16)] = z16
    return carry

  lax.fori_loop(0, CHUNK, _zero_rows, 0)

  row0 = s * ROWS_PER_TILE
  for off in range(0, ROWS_PER_TILE, CHUNK):
    sz = min(CHUNK, ROWS_PER_TILE - off)
    pltpu.sync_copy(msg0.at[pl.ds(0, sz)], acc_u.at[pl.ds(row0 + off, sz)])
  drow0 = s * DROWS_PER_TILE
  for off in range(0, DROWS_PER_TILE, CHUNK):
    sz = min(CHUNK, DROWS_PER_TILE - off)
    pltpu.sync_copy(msg0.at[pl.ds(0, sz)], acc_d.at[pl.ds(drow0 + off, sz)])
  plsc.subcore_barrier()

  edge0 = wid * CHUNKS_PER_TILE * CHUNK
  iota16 = lax.iota(jnp.int32, 16)

  def _fetch(ch, p):
    off = edge0 + ch * CHUNK
    pltpu.sync_copy(src_hbm.at[pl.ds(off, CHUNK)], sidx[p])
    pltpu.sync_copy(dst_hbm.at[pl.ds(off, CHUNK)], didx[p])
    pltpu.async_copy(feat_hbm.at[sidx[p]], msg[p], gsem[p])
    pltpu.async_copy(elx_hbm.at[sidx[p]], elg[p], gsem[p])
    pltpu.async_copy(erx_hbm.at[didx[p]], erg[p], gsem[p])

  def _wait_fetch(p):
    pltpu.make_async_copy(feat_hbm.at[sidx[p]], msg[p], gsem[p]).wait()
    pltpu.make_async_copy(elx_hbm.at[sidx[p]], elg[p], gsem[p]).wait()
    pltpu.make_async_copy(erx_hbm.at[didx[p]], erg[p], gsem[p]).wait()

  def _compute(p):
    msg_j, elg_j, erg_j, didx_j = msg[p], elg[p], erg[p], didx[p]

    def _edge(i2, inner):
      for u in range(2):
        i = i2 * 2 + u
        sv = elg_j[i, pl.ds(0, 16)] + erg_j[i, pl.ds(0, 16)]
        e = jnp.maximum(sv, 0.2 * sv)
        ee = jnp.exp(e)
        eeb[i] = ee
        for h in range(H):
          idx = jnp.full((16,), h, dtype=jnp.int32)
          bc = jnp.take_along_axis(
              ee, idx, axis=0,
              mode=lax.GatherScatterMode.PROMISE_IN_BOUNDS)
          fsl = pl.ds(h * 16, 16)
          msg_j[i, fsl] = bc * msg_j[i, fsl]
      return inner

    lax.fori_loop(0, CHUNK // 2, _edge, 0)

    for k in range(CHUNK // 16):
      rows = iota16 + (k * 16)
      dv = didx_j[pl.ds(k * 16, 16)]
      didx8[pl.ds(k * 16, 16)] = lax.shift_right_logical(dv, 3)
      colb = (dv & 7) * 16
      for h in range(H):
        vals = plsc.load_gather(
            eeb, [rows, jnp.full((16,), h, dtype=jnp.int32)])
        plsc.store_scatter(eep, [rows, colb + h], vals)

  def _scatter_and_rezero(p):
    pltpu.async_copy(msg[p], acc_u.at[didx[p]], usem, add=True)
    pltpu.async_copy(eep, acc_d.at[didx8], dsem, add=True).wait()
    didx_j = didx[p]
    for k in range(CHUNK // 16):
      rows = iota16 + (k * 16)
      dv = didx_j[pl.ds(k * 16, 16)]
      colb = (dv & 7) * 16
      for h in range(H):
        plsc.store_scatter(eep, [rows, colb + h], z16)

  def _wait_u(p):
    pltpu.make_async_copy(msg[p], acc_u.at[didx[p]], usem).wait()

  _fetch(jnp.int32(0), 0)

  def _body(g2, carry):
    ch0 = g2 * 2
    _fetch(ch0 + 1, 1)
    _wait_fetch(0)
    _compute(0)
    _scatter_and_rezero(0)
    _wait_u(0)
    _fetch(jnp.minimum(ch0 + 2, CHUNKS_PER_TILE - 2), 0)
    _wait_fetch(1)
    _compute(1)
    _scatter_and_rezero(1)
    _wait_u(1)
    return carry

  lax.fori_loop(0, CHUNKS_PER_TILE // 2, _body, 0)
  _wait_fetch(0)

  plsc.subcore_barrier()
  rows = pl.ds(row0, ROWS_PER_TILE)
  pltpu.sync_copy(acc_u.at[rows], out_u.at[c, rows])
  drows = pl.ds(drow0, DROWS_PER_TILE)
  pltpu.sync_copy(acc_d.at[drows], out_d.at[c, drows])


def _edge_pass(srcf, dstf, feat, elx, erx):
  mesh = plsc.VectorSubcoreMesh(core_axis_name="c", subcore_axis_name="s")
  idx_t = pltpu.VMEM((CHUNK,), jnp.int32)
  row_t = pltpu.VMEM((CHUNK, 128), jnp.float32)
  ee_t = pltpu.VMEM((CHUNK, 16), jnp.float32)
  return pl.kernel(
      _edge_body,
      out_type=[
          jax.ShapeDtypeStruct((2, NP, 128), jnp.float32),
          jax.ShapeDtypeStruct((2, NP // 8, 128), jnp.float32),
      ],
      mesh=mesh,
      compiler_params=pltpu.CompilerParams(needs_layout_passes=False),
      scratch_types=[
          pltpu.VMEM_SHARED((NP, 128), jnp.float32),
          pltpu.VMEM_SHARED((NP // 8, 128), jnp.float32),
          idx_t, idx_t,
          idx_t, idx_t,
          idx_t,
          row_t, row_t,
          row_t, row_t,
          row_t, row_t,
          ee_t,
          row_t,
          pltpu.SemaphoreType.DMA,
          pltpu.SemaphoreType.DMA,
          pltpu.SemaphoreType.DMA,
          pltpu.SemaphoreType.DMA,
      ],
  )(srcf, dstf, feat, elx, erx)


def _attn_mats(al, ar):
  a = al.reshape(H, F)
  r = ar.reshape(H, F)
  eye = jnp.eye(H, dtype=jnp.float32)
  cl = (a[:, :, None] * eye[:, None, :]).reshape(H * F, H)
  cr = (r[:, :, None] * eye[:, None, :]).reshape(H * F, H)
  c1 = jnp.concatenate([cl, cr], axis=1)
  c2 = jnp.concatenate([cr, cl], axis=1)
  return c1, c2


@jax.jit
def kernel(x, edge_index, mask, W1, al1, ar1, b1, W2, al2, ar2, b2):
  xp = jnp.pad(x, ((0, NP - N), (0, 0)))
  src = edge_index[0]
  dst = edge_index[1]
  npad = EP - E
  pad_idx = N + (jnp.arange(npad, dtype=jnp.int32) % (NP - N))
  srcf = jnp.concatenate([src, pad_idx])
  dstf = jnp.concatenate([dst, pad_idx])

  w1t = W1.T
  w2t = W2.T
  c11, c12 = _attn_mats(al1, ar1)
  c21, c22 = _attn_mats(al2, ar2)
  b16 = jnp.concatenate(
      [jnp.kron(jnp.eye(8, dtype=jnp.float32), jnp.ones((1, 16), jnp.float32)),
       jnp.zeros((8, 128), jnp.float32)], axis=0)
  bias1 = b1.reshape(1, 128)
  bias2 = b2.reshape(1, 128)
  mask2d = jnp.pad(mask, (0, NP - N)).reshape(NP, 1)

  feat1, elx1, erx1 = _tc_layer1(xp, w1t, c11, c12)
  u1, d1p = _edge_pass(srcf, dstf, feat1, elx1, erx1)
  d1 = d1p.reshape(2, NP, 16)

  feat2, elx2, erx2 = _tc_mid(u1[0], u1[1], d1[0], d1[1], b16, bias1,
                              w2t, c21, c22)
  u2, d2p = _edge_pass(srcf, dstf, feat2, elx2, erx2)
  d2 = d2p.reshape(2, NP, 16)

  res = _tc_final(u2[0], u2[1], d2[0], d2[1], b16, bias2, mask2d)
  return res[0, 0]
